# baseline (speedup 1.0000x reference)
import functools

import jax
import jax.numpy as jnp
from jax import lax
from jax.experimental import pallas as pl
from jax.experimental.pallas import tpu as pltpu
from jax.experimental.pallas import tpu_sc as plsc

VOCAB = 1000000
EMBED_DIM = 32
BATCH = 16384
HIST = 50
B = BATCH * HIST

NUM_CORES = 2
NUM_SUBCORES = 16
NW = NUM_CORES * NUM_SUBCORES
BW = BATCH // NW
IDXW = BW * HIST
NBT = BW // 128
NJT = EMBED_DIM // 8


def _gather_body(idx_hbm, table_hbm, o5_hbm,
                 idxblk_v, gidx_v, rows_v, tiles_v, gsem, wsem):
    wid = lax.axis_index("s") * NUM_CORES + lax.axis_index("c")
    tb0 = wid * NBT
    iota = lax.iota(jnp.int32, 16)
    iota50 = iota * HIST

    pltpu.sync_copy(idx_hbm.at[pl.ds(wid * IDXW, IDXW)], idxblk_v)

    def _extract_and_fire(h, off):
        for k in range(BW // 16):
            ids = iota50 + (k * 16 * HIST + h)
            vals = plsc.load_gather(idxblk_v, [ids])
            gidx_v[pl.ds(off + k * 16, 16)] = vals
        pltpu.async_copy(
            table_hbm.at[gidx_v.at[pl.ds(off, BW)]],
            rows_v.at[pl.ds(off, BW)], gsem)

    _extract_and_fire(0, 0)

    @pl.loop(0, HIST)
    def _h_step(h):
        b = lax.rem(h, 2)
        nb = 1 - b
        roff = b * BW

        pltpu.make_async_copy(
            table_hbm.at[gidx_v.at[pl.ds(roff, BW)]],
            rows_v.at[pl.ds(roff, BW)], gsem).wait()

        @pl.when(h + 1 < HIST)
        def _():
            _extract_and_fire(h + 1, nb * BW)

        lat_r = lax.shift_right_logical(iota, 2)
        lat_c = lax.bitwise_and(iota, 3)
        zero16 = jnp.zeros((16,), jnp.int32)

        @pl.loop(0, NJT * NBT)
        def _tile(m):
            tj = m // NBT
            tb = lax.rem(m, NBT)
            vb = zero16 + b
            vtj = zero16 + tj
            vtb = zero16 + tb
            base_r = lat_r + (roff + tb * 128)
            for jb in range(2):
                cvec = lat_c + (tj * 8 + jb * 4)
                i3 = (lat_c + jb * 4) * 128 + lat_r
                for k4 in range(32):
                    rvec = base_r + k4 * 4
                    vals = plsc.load_gather(rows_v, [rvec, cvec])
                    plsc.store_scatter(
                        tiles_v, [vb, vtj, vtb, i3 + k4 * 4], vals)

        @pl.when(h >= 1)
        def _():
            pltpu.make_async_copy(
                tiles_v.at[0], o5_hbm.at[0, :, pl.ds(0, NBT)], wsem).wait()

        pltpu.async_copy(
            tiles_v.at[b], o5_hbm.at[h, :, pl.ds(tb0, NBT)], wsem)

    pltpu.make_async_copy(
        tiles_v.at[0], o5_hbm.at[0, :, pl.ds(0, NBT)], wsem).wait()


DT_ROWS = 256
DT_CHUNKS = 122
DT_TAIL = VOCAB - NW * DT_CHUNKS * DT_ROWS


def _detile_body(tab_hbm, out_hbm, buf_v, obuf_v, isem, osem):
    wid = lax.axis_index("s") * NUM_CORES + lax.axis_index("c")
    base = wid * DT_CHUNKS * DT_ROWS

    def _stage(c, sel):
        r0 = pl.multiple_of(base + c * DT_ROWS, 256)
        pltpu.async_copy(
            tab_hbm.at[pl.ds(r0, DT_ROWS), :],
            buf_v.at[sel, pl.ds(0, DT_ROWS)], isem)

    _stage(0, 0)

    @pl.loop(0, DT_CHUNKS)
    def _chunk(c):
        sel = lax.rem(c, 2)
        nsel = 1 - sel
        pltpu.make_async_copy(
            tab_hbm.at[pl.ds(0, DT_ROWS), :],
            buf_v.at[0, pl.ds(0, DT_ROWS)], isem).wait()

        @pl.when(c + 1 < DT_CHUNKS)
        def _():
            _stage(c + 1, nsel)

        @pl.loop(0, DT_ROWS // 4)
        def _t(t):
            for r4 in range(4):
                r = t * 4 + r4
                for half in range(2):
                    v = buf_v[sel, r, pl.ds(half * 16, 16)]
                    obuf_v[sel, t, pl.ds(r4 * 32 + half * 16, 16)] = v

        @pl.when(c >= 1)
        def _():
            pltpu.make_async_copy(
                obuf_v.at[0], out_hbm.at[pl.ds(0, DT_ROWS // 4), :],
                osem).wait()
        pltpu.async_copy(
            obuf_v.at[sel],
            out_hbm.at[pl.ds(
                pl.multiple_of((base + c * DT_ROWS) // 4, 64),
                DT_ROWS // 4), :],
            osem)

    pltpu.make_async_copy(
        obuf_v.at[0], out_hbm.at[pl.ds(0, DT_ROWS // 4), :], osem).wait()

    @pl.when(wid == 0)
    def _tail():
        t0 = NW * DT_CHUNKS * DT_ROWS
        for r0, nr in ((t0, 256), (t0 + 256, 256), (t0 + 512, 64)):
            pltpu.sync_copy(tab_hbm.at[pl.ds(r0, nr), :],
                            buf_v.at[0, pl.ds(0, nr)])

            @pl.loop(0, nr // 4)
            def _t(t):
                for r4 in range(4):
                    r = t * 4 + r4
                    for half in range(2):
                        v = buf_v[0, r, pl.ds(half * 16, 16)]
                        obuf_v[0, t, pl.ds(r4 * 32 + half * 16, 16)] = v
            pltpu.sync_copy(obuf_v.at[0, pl.ds(0, nr // 4)],
                            out_hbm.at[pl.ds(r0 // 4, nr // 4), :])


def _build_detile():
    mesh = plsc.VectorSubcoreMesh(
        core_axis_name="c", subcore_axis_name="s",
        num_cores=NUM_CORES, num_subcores=NUM_SUBCORES)
    return pl.kernel(
        _detile_body,
        out_type=jax.ShapeDtypeStruct((VOCAB // 4, 128), jnp.float32),
        mesh=mesh,
        scratch_types=[
            pltpu.VMEM((2, DT_ROWS, EMBED_DIM), jnp.float32),
            pltpu.VMEM((2, DT_ROWS // 4, 128), jnp.float32),
            pltpu.SemaphoreType.DMA,
            pltpu.SemaphoreType.DMA,
        ],
        compiler_params=pltpu.CompilerParams(use_tc_tiling_on_sc=True),
    )


def _build_kernel():
    mesh = plsc.VectorSubcoreMesh(
        core_axis_name="c", subcore_axis_name="s",
        num_cores=NUM_CORES, num_subcores=NUM_SUBCORES)
    return pl.kernel(
        _gather_body,
        out_type=jax.ShapeDtypeStruct((HIST, NJT, BATCH // 128, 1024),
                                      jnp.float32),
        mesh=mesh,
        scratch_types=[
            pltpu.VMEM((IDXW,), jnp.int32),
            pltpu.VMEM((2 * BW,), jnp.int32),
            pltpu.VMEM((2 * BW, EMBED_DIM), jnp.float32),
            pltpu.VMEM((2, NJT, NBT, 1024), jnp.float32),
            pltpu.SemaphoreType.DMA,
            pltpu.SemaphoreType.DMA,
        ],
        compiler_params=pltpu.CompilerParams(
            use_tc_tiling_on_sc=False, needs_layout_passes=False),
    )


def kernel(x, table):
    idx = x.reshape(-1).astype(jnp.int32)
    tab_lin = _build_detile()(table).reshape(VOCAB, EMBED_DIM)
    o5 = _build_kernel()(idx, tab_lin)
    o6 = o5.reshape(HIST, NJT, BATCH // 128, 8, 128)
    out = o6.transpose(2, 4, 0, 1, 3).reshape(BATCH, HIST, EMBED_DIM)
    return out


# speedup vs baseline: 1.0740x; 1.0740x over previous
import functools

import jax
import jax.numpy as jnp
from jax import lax
from jax.experimental import pallas as pl
from jax.experimental.pallas import tpu as pltpu
from jax.experimental.pallas import tpu_sc as plsc

VOCAB = 1000000
EMBED_DIM = 32
BATCH = 16384
HIST = 50
B = BATCH * HIST

NUM_CORES = 2
NUM_SUBCORES = 16
NW = NUM_CORES * NUM_SUBCORES
BW = BATCH // NW
IDXW = BW * HIST
NBT = BW // 128
NJT = EMBED_DIM // 8


def _gather_body(idx_hbm, table_hbm, o5_hbm,
                 idxblk_v, gidx_v, rows_v, tiles_v, gsem, wsem):
    wid = lax.axis_index("s") * NUM_CORES + lax.axis_index("c")
    tb0 = wid * NBT
    iota = lax.iota(jnp.int32, 16)
    iota50 = iota * HIST

    pltpu.sync_copy(idx_hbm.at[pl.ds(wid * IDXW, IDXW)], idxblk_v)

    def _extract_and_fire(h, off):
        for k in range(BW // 16):
            ids = iota50 + (k * 16 * HIST + h)
            vals = plsc.load_gather(idxblk_v, [ids])
            gidx_v[pl.ds(off + k * 16, 16)] = vals
        pltpu.async_copy(
            table_hbm.at[gidx_v.at[pl.ds(off, BW)]],
            rows_v.at[pl.ds(off, BW)], gsem)

    _extract_and_fire(0, 0)

    @pl.loop(0, HIST)
    def _h_step(h):
        b = lax.rem(h, 2)
        nb = 1 - b
        roff = b * BW

        pltpu.make_async_copy(
            table_hbm.at[gidx_v.at[pl.ds(roff, BW)]],
            rows_v.at[pl.ds(roff, BW)], gsem).wait()

        @pl.when(h + 1 < HIST)
        def _():
            _extract_and_fire(h + 1, nb * BW)

        lat_r = lax.shift_right_logical(iota, 2)
        lat_c = lax.bitwise_and(iota, 3)
        zero16 = jnp.zeros((16,), jnp.int32)

        @pl.loop(0, NJT * NBT)
        def _tile(m):
            tj = m // NBT
            tb = lax.rem(m, NBT)
            vb = zero16 + b
            vtj = zero16 + tj
            vtb = zero16 + tb
            base_r = lat_r + (roff + tb * 128)
            for jb in range(2):
                cvec = lat_c + (tj * 8 + jb * 4)
                i3 = (lat_c + jb * 4) * 128 + lat_r
                for k4 in range(32):
                    rvec = base_r + k4 * 4
                    vals = plsc.load_gather(rows_v, [rvec, cvec])
                    plsc.store_scatter(
                        tiles_v, [vb, vtj, vtb, i3 + k4 * 4], vals)

        @pl.when(h >= 1)
        def _():
            pltpu.make_async_copy(
                tiles_v.at[0], o5_hbm.at[0, :, pl.ds(0, NBT)], wsem).wait()

        pltpu.async_copy(
            tiles_v.at[b], o5_hbm.at[h, :, pl.ds(tb0, NBT)], wsem)

    pltpu.make_async_copy(
        tiles_v.at[0], o5_hbm.at[0, :, pl.ds(0, NBT)], wsem).wait()


def _build_kernel():
    mesh = plsc.VectorSubcoreMesh(
        core_axis_name="c", subcore_axis_name="s",
        num_cores=NUM_CORES, num_subcores=NUM_SUBCORES)
    return pl.kernel(
        _gather_body,
        out_type=jax.ShapeDtypeStruct((HIST, NJT, BATCH // 128, 1024),
                                      jnp.float32),
        mesh=mesh,
        scratch_types=[
            pltpu.VMEM((IDXW,), jnp.int32),
            pltpu.VMEM((2 * BW,), jnp.int32),
            pltpu.VMEM((2 * BW, EMBED_DIM), jnp.float32),
            pltpu.VMEM((2, NJT, NBT, 1024), jnp.float32),
            pltpu.SemaphoreType.DMA,
            pltpu.SemaphoreType.DMA,
        ],
        compiler_params=pltpu.CompilerParams(
            use_tc_tiling_on_sc=False, needs_layout_passes=False),
    )


def kernel(x, table):
    idx = x.reshape(-1).astype(jnp.int32)
    o5 = _build_kernel()(idx, table)
    o6 = o5.reshape(HIST, NJT, BATCH // 128, 8, 128)
    out = o6.transpose(2, 4, 0, 1, 3).reshape(BATCH, HIST, EMBED_DIM)
    return out


# speedup vs baseline: 1.3295x; 1.2379x over previous
import functools

import jax
import jax.numpy as jnp
from jax import lax
from jax.experimental import pallas as pl
from jax.experimental.pallas import tpu as pltpu
from jax.experimental.pallas import tpu_sc as plsc

VOCAB = 1000000
EMBED_DIM = 32
BATCH = 16384
HIST = 50
B = BATCH * HIST

NUM_CORES = 2
NUM_SUBCORES = 16
NW = NUM_CORES * NUM_SUBCORES
BW = BATCH // NW
IDXW = BW * HIST
NBT = BW // 128
NJT = EMBED_DIM // 8


def _gather_body(idx_hbm, table_hbm, o5_hbm,
                 idxblk_v, gidx_v, rows_v, tiles_v, gsem, wsem):
    wid = lax.axis_index("s") * NUM_CORES + lax.axis_index("c")
    tb0 = wid * NBT
    iota = lax.iota(jnp.int32, 16)
    iota50 = iota * HIST
    rot = [lax.bitwise_and(iota + d, 15) for d in range(16)]

    pltpu.sync_copy(idx_hbm.at[pl.ds(wid * IDXW, IDXW)], idxblk_v)

    def _extract_and_fire(h, off):
        for k in range(BW // 16):
            ids = iota50 + (k * 16 * HIST + h)
            vals = plsc.load_gather(idxblk_v, [ids])
            gidx_v[pl.ds(off + k * 16, 16)] = vals
        pltpu.async_copy(
            table_hbm.at[gidx_v.at[pl.ds(off, BW)]],
            rows_v.at[pl.ds(off, BW)], gsem)

    _extract_and_fire(0, 0)

    @pl.loop(0, HIST)
    def _h_step(h):
        b = lax.rem(h, 2)
        nb = 1 - b
        roff = b * BW

        pltpu.make_async_copy(
            table_hbm.at[gidx_v.at[pl.ds(roff, BW)]],
            rows_v.at[pl.ds(roff, BW)], gsem).wait()

        @pl.when(h + 1 < HIST)
        def _():
            _extract_and_fire(h + 1, nb * BW)

        zero16 = jnp.zeros((16,), jnp.int32)

        @pl.loop(0, BW // 16)
        def _rblk(R):
            r0loc = R * 16
            tb = lax.shift_right_logical(R, 3)
            vb = zero16 + b
            vtb = zero16 + tb
            lanevec = iota + lax.bitwise_and(r0loc, 127)
            rvec = iota + (roff + r0loc)
            for C in range(2):
                for d in range(16):
                    cvec = rot[d] + C * 16
                    vals = plsc.load_gather(rows_v, [rvec, cvec])
                    vtj = lax.shift_right_logical(cvec, 3)
                    i3 = lax.bitwise_and(cvec, 7) * 128 + lanevec
                    plsc.store_scatter(tiles_v, [vb, vtj, vtb, i3], vals)

        @pl.when(h >= 1)
        def _():
            pltpu.make_async_copy(
                tiles_v.at[0], o5_hbm.at[0, :, pl.ds(0, NBT)], wsem).wait()

        pltpu.async_copy(
            tiles_v.at[b], o5_hbm.at[h, :, pl.ds(tb0, NBT)], wsem)

    pltpu.make_async_copy(
        tiles_v.at[0], o5_hbm.at[0, :, pl.ds(0, NBT)], wsem).wait()


def _build_kernel():
    mesh = plsc.VectorSubcoreMesh(
        core_axis_name="c", subcore_axis_name="s",
        num_cores=NUM_CORES, num_subcores=NUM_SUBCORES)
    return pl.kernel(
        _gather_body,
        out_type=jax.ShapeDtypeStruct((HIST, NJT, BATCH // 128, 1024),
                                      jnp.float32),
        mesh=mesh,
        scratch_types=[
            pltpu.VMEM((IDXW,), jnp.int32),
            pltpu.VMEM((2 * BW,), jnp.int32),
            pltpu.VMEM((2 * BW, EMBED_DIM), jnp.float32),
            pltpu.VMEM((2, NJT, NBT, 1024), jnp.float32),
            pltpu.SemaphoreType.DMA,
            pltpu.SemaphoreType.DMA,
        ],
        compiler_params=pltpu.CompilerParams(
            use_tc_tiling_on_sc=False, needs_layout_passes=False),
    )


def kernel(x, table):
    idx = x.reshape(-1).astype(jnp.int32)
    o5 = _build_kernel()(idx, table)
    o6 = o5.reshape(HIST, NJT, BATCH // 128, 8, 128)
    out = o6.transpose(2, 4, 0, 1, 3).reshape(BATCH, HIST, EMBED_DIM)
    return out


# speedup vs baseline: 1.3323x; 1.0021x over previous
import functools

import jax
import jax.numpy as jnp
from jax import lax
from jax.experimental import pallas as pl
from jax.experimental.pallas import tpu as pltpu
from jax.experimental.pallas import tpu_sc as plsc

VOCAB = 1000000
EMBED_DIM = 32
BATCH = 16384
HIST = 50
B = BATCH * HIST

NUM_CORES = 2
NUM_SUBCORES = 16
NW = NUM_CORES * NUM_SUBCORES
BW = BATCH // NW
IDXW = BW * HIST
NBT = BW // 128
NJT = EMBED_DIM // 8


def _gather_body(idx_hbm, table_hbm, o5_hbm,
                 idxblk_v, gidx_v, rows_v, tiles_v, gsem, wsem):
    wid = lax.axis_index("s") * NUM_CORES + lax.axis_index("c")
    tb0 = wid * NBT
    iota = lax.iota(jnp.int32, 16)
    iota50 = iota * HIST
    rot = [lax.bitwise_and(iota + d, 15) for d in range(16)]

    pltpu.sync_copy(idx_hbm.at[pl.ds(wid * IDXW, IDXW)], idxblk_v)

    def _extract_and_fire(h, off):
        for k in range(BW // 16):
            ids = iota50 + (k * 16 * HIST + h)
            vals = plsc.load_gather(idxblk_v, [ids])
            gidx_v[pl.ds(off + k * 16, 16)] = vals
        pltpu.async_copy(
            table_hbm.at[gidx_v.at[pl.ds(off, BW)]],
            rows_v.at[pl.ds(off, BW)], gsem)

    _extract_and_fire(0, 0)

    @pl.loop(0, HIST)
    def _h_step(h):
        b = lax.rem(h, 2)
        nb = 1 - b
        roff = b * BW

        @pl.when(h + 1 < HIST)
        def _():
            _extract_and_fire(h + 1, nb * BW)

        pltpu.make_async_copy(
            table_hbm.at[gidx_v.at[pl.ds(roff, BW)]],
            rows_v.at[pl.ds(roff, BW)], gsem).wait()

        zero16 = jnp.zeros((16,), jnp.int32)

        @pl.loop(0, BW // 16)
        def _rblk(R):
            r0loc = R * 16
            tb = lax.shift_right_logical(R, 3)
            vb = zero16 + b
            vtb = zero16 + tb
            lanevec = iota + lax.bitwise_and(r0loc, 127)
            rvec = iota + (roff + r0loc)
            for C in range(2):
                for d in range(16):
                    cvec = rot[d] + C * 16
                    vals = plsc.load_gather(rows_v, [rvec, cvec])
                    vtj = lax.shift_right_logical(cvec, 3)
                    i3 = lax.bitwise_and(cvec, 7) * 128 + lanevec
                    plsc.store_scatter(tiles_v, [vb, vtj, vtb, i3], vals)

        @pl.when(h >= 1)
        def _():
            pltpu.make_async_copy(
                tiles_v.at[0], o5_hbm.at[0, :, pl.ds(0, NBT)], wsem).wait()

        pltpu.async_copy(
            tiles_v.at[b], o5_hbm.at[h, :, pl.ds(tb0, NBT)], wsem)

    pltpu.make_async_copy(
        tiles_v.at[0], o5_hbm.at[0, :, pl.ds(0, NBT)], wsem).wait()


def _build_kernel():
    mesh = plsc.VectorSubcoreMesh(
        core_axis_name="c", subcore_axis_name="s",
        num_cores=NUM_CORES, num_subcores=NUM_SUBCORES)
    return pl.kernel(
        _gather_body,
        out_type=jax.ShapeDtypeStruct((HIST, NJT, BATCH // 128, 1024),
                                      jnp.float32),
        mesh=mesh,
        scratch_types=[
            pltpu.VMEM((IDXW,), jnp.int32),
            pltpu.VMEM((2 * BW,), jnp.int32),
            pltpu.VMEM((2 * BW, EMBED_DIM), jnp.float32),
            pltpu.VMEM((2, NJT, NBT, 1024), jnp.float32),
            pltpu.SemaphoreType.DMA,
            pltpu.SemaphoreType.DMA,
        ],
        compiler_params=pltpu.CompilerParams(
            use_tc_tiling_on_sc=False, needs_layout_passes=False),
    )


def kernel(x, table):
    idx = x.reshape(-1).astype(jnp.int32)
    o5 = _build_kernel()(idx, table)
    o6 = o5.reshape(HIST, NJT, BATCH // 128, 8, 128)
    out = o6.transpose(2, 4, 0, 1, 3).reshape(BATCH, HIST, EMBED_DIM)
    return out
